# baseline (speedup 1.0000x reference)
import functools
import math

import jax
import jax.numpy as jnp
from jax import lax
from jax.experimental import pallas as pl
from jax.experimental.pallas import tpu as pltpu
from jax.experimental.pallas import tpu_sc as plsc

DIM = 64
N_ROUNDS = 8
L_SIZE = 100000
C_SIZE = 40000

NC = 2
NS = 16
EB = 128
KB = 8
SWEEP = NS * KB * EB
ZROWS = 128

C_NQ, C_QSIZE, C_QPAD = 4, 10016, 10112
L_NQ, L_QSIZE, L_QPAD = 8, 12512, 12544


def _make_sc_aggregate(e_rows, n_src, s_out, n_q, qsize, qs_pad):
    stripe_z = qs_pad // NS
    n_zfull = stripe_z // ZROWS
    z_rem = stripe_z - n_zfull * ZROWS
    mesh = plsc.VectorSubcoreMesh(
        core_axis_name="c", subcore_axis_name="s",
        num_cores=NC, num_subcores=NS)

    out_type = jax.ShapeDtypeStruct((s_out, DIM), jnp.float32)
    scratch = [
        pltpu.VMEM_SHARED((qs_pad, DIM), jnp.float32),
        pltpu.VMEM((32,), jnp.int32),
        pltpu.VMEM((KB, EB), jnp.int32),
        pltpu.VMEM((KB, EB), jnp.int32),
        pltpu.VMEM((KB * EB, DIM), jnp.float32),
        pltpu.VMEM((ZROWS, DIM), jnp.float32),
        pltpu.SemaphoreType.DMA,
        pltpu.SemaphoreType.DMA,
    ]

    def body(src2d, *rest):
        dsts = rest[:n_q]
        meta, msg, zhbm, out = rest[n_q:n_q + 4]
        acc, meta_v, sidx, didx, rows, zbuf, semg, sems = rest[n_q + 4:]
        cid = lax.axis_index("c")
        sid = lax.axis_index("s")
        pltpu.sync_copy(zhbm, zbuf)
        pltpu.sync_copy(meta, meta_v)

        for p in range(n_q // NC):
            for c in range(NC):
                q = NC * p + c

                @pl.when(cid == c)
                def _(q=q):
                    zb = sid * stripe_z
                    for t in range(n_zfull):
                        pltpu.sync_copy(zbuf,
                                        acc.at[pl.ds(zb + t * ZROWS, ZROWS)])
                    if z_rem:
                        pltpu.sync_copy(zbuf.at[pl.ds(0, z_rem)],
                                        acc.at[pl.ds(zb + n_zfull * ZROWS,
                                                     z_rem)])
                    plsc.subcore_barrier()

                    mv = meta_v[pl.ds(0, 16)]
                    a0 = mv[2 * q]
                    ni = mv[2 * q + 1]

                    def it(i, cc):
                        r0 = a0 + (sid + i * NS) * KB
                        pltpu.sync_copy(src2d.at[pl.ds(r0, KB)], sidx)
                        pltpu.sync_copy(dsts[q].at[pl.ds(r0, KB)], didx)
                        for k in range(KB):
                            pltpu.async_copy(msg.at[sidx.at[k]],
                                             rows.at[pl.ds(k * EB, EB)],
                                             semg)
                        pltpu.make_async_copy(msg.at[pl.ds(0, KB * EB)],
                                             rows, semg).wait()
                        for k in range(KB):
                            pltpu.async_copy(rows.at[pl.ds(k * EB, EB)],
                                             acc.at[didx.at[k]], sems,
                                             add=True)
                        pltpu.make_async_copy(rows,
                                             acc.at[pl.ds(0, KB * EB)],
                                             sems).wait()
                        return cc
                    lax.fori_loop(0, ni, it, 0)
                    plsc.subcore_barrier()

                    rows_out = min(qsize, s_out - q * qsize)
                    so = (rows_out // (NS * 8)) * 8
                    orem = rows_out - NS * so
                    cb = sid * so
                    pltpu.sync_copy(acc.at[pl.ds(cb, so)],
                                    out.at[pl.ds(q * qsize + cb, so)])
                    if orem:
                        @pl.when(sid == 0)
                        def _():
                            pltpu.sync_copy(
                                acc.at[pl.ds(NS * so, orem)],
                                out.at[pl.ds(q * qsize + NS * so, orem)])
                    plsc.subcore_barrier()

    return pl.kernel(body, out_type=out_type, mesh=mesh,
                     scratch_types=scratch,
                     compiler_params=pltpu.CompilerParams(
                         use_tc_tiling_on_sc=False))


def _ln(x, g, b, eps=1e-5):
    mu = jnp.mean(x, axis=1, keepdims=True)
    var = jnp.mean((x - mu) ** 2, axis=1, keepdims=True)
    return (x - mu) * lax.rsqrt(var + eps) * g + b


def _mlp3(x, W0, b0, W1, b1, W2, b2):
    h = jnp.maximum(jnp.dot(x, W0, preferred_element_type=jnp.float32) + b0, 0.0)
    h = jnp.maximum(jnp.dot(h, W1, preferred_element_type=jnp.float32) + b1, 0.0)
    return jnp.dot(h, W2, preferred_element_type=jnp.float32) + b2


def _swap_pairs(x):
    r = x.shape[0]
    nxt = jnp.concatenate([x[1:], x[:1]], axis=0)
    prv = jnp.concatenate([x[-1:], x[:-1]], axis=0)
    row = lax.broadcasted_iota(jnp.int32, (r, 1), 0)
    return jnp.where(row % 2 == 0, nxt, prv)


def _sigmoid(x):
    return 1.0 / (1.0 + jnp.exp(-x))


def _lstm_block(x, h, c, W_ih, W_hh, g_ih, b_ih, g_hh, b_hh, g_c, b_c):
    gi = _ln(jnp.dot(x, W_ih, preferred_element_type=jnp.float32), g_ih, b_ih)
    gh = _ln(jnp.dot(h, W_hh, preferred_element_type=jnp.float32), g_hh, b_hh)
    gates = gi + gh
    i_g = gates[:, 0:DIM]
    f_g = gates[:, DIM:2 * DIM]
    g_g = gates[:, 2 * DIM:3 * DIM]
    o_g = gates[:, 3 * DIM:4 * DIM]
    c_new = _sigmoid(f_g) * c + _sigmoid(i_g) * jnp.tanh(g_g)
    h_new = _sigmoid(o_g) * jnp.tanh(_ln(c_new, g_c, b_c))
    return h_new, c_new


_FULL2 = lambda shape: pl.BlockSpec(shape, lambda i: (0, 0))


def _make_lstm_c(R):
    nb = C_SIZE // R

    def body(a, h, c, W_ih, W_hh, g_ih, b_ih, g_hh, b_hh, g_c, b_c,
             W0, b0, W1, b1, W2, b2, oh, oc, om):
        h_new, c_new = _lstm_block(
            a[...], h[...], c[...], W_ih[...], W_hh[...], g_ih[...],
            b_ih[...], g_hh[...], b_hh[...], g_c[...], b_c[...])
        oh[...] = h_new
        oc[...] = c_new
        om[...] = _mlp3(h_new, W0[...], b0[...], W1[...], b1[...],
                        W2[...], b2[...])

    row = lambda shape: pl.BlockSpec(shape, lambda i: (i, 0))
    in_specs = [row((R, DIM)), row((R, DIM)), row((R, DIM)),
                _FULL2((DIM, 4 * DIM)), _FULL2((DIM, 4 * DIM)),
                _FULL2((1, 4 * DIM)), _FULL2((1, 4 * DIM)),
                _FULL2((1, 4 * DIM)), _FULL2((1, 4 * DIM)),
                _FULL2((1, DIM)), _FULL2((1, DIM)),
                _FULL2((DIM, DIM)), _FULL2((1, DIM)),
                _FULL2((DIM, DIM)), _FULL2((1, DIM)),
                _FULL2((DIM, DIM)), _FULL2((1, DIM))]
    out_specs = [row((R, DIM))] * 3
    out_shape = [jax.ShapeDtypeStruct((C_SIZE, DIM), jnp.float32)] * 3
    return pl.pallas_call(body, grid=(nb,), in_specs=in_specs,
                          out_specs=out_specs, out_shape=out_shape)


def _make_lstm_l(R):
    nb = L_SIZE // R

    def body(a, h, c, W_ih, W_hh, g_ih, b_ih, g_hh, b_hh,
             g_c, b_c, W0, b0, W1, b1, W2, b2, oh, oc, om):
        hv = h[...]
        x = jnp.concatenate([a[...], _swap_pairs(hv)], axis=1)
        h_new, c_new = _lstm_block(
            x, hv, c[...], W_ih[...], W_hh[...], g_ih[...], b_ih[...],
            g_hh[...], b_hh[...], g_c[...], b_c[...])
        oh[...] = h_new
        oc[...] = c_new
        om[...] = _mlp3(h_new, W0[...], b0[...], W1[...], b1[...],
                        W2[...], b2[...])

    row = lambda shape: pl.BlockSpec(shape, lambda i: (i, 0))
    in_specs = [row((R, DIM)), row((R, DIM)), row((R, DIM)),
                _FULL2((2 * DIM, 4 * DIM)), _FULL2((DIM, 4 * DIM)),
                _FULL2((1, 4 * DIM)), _FULL2((1, 4 * DIM)),
                _FULL2((1, 4 * DIM)), _FULL2((1, 4 * DIM)),
                _FULL2((1, DIM)), _FULL2((1, DIM)),
                _FULL2((DIM, DIM)), _FULL2((1, DIM)),
                _FULL2((DIM, DIM)), _FULL2((1, DIM)),
                _FULL2((DIM, DIM)), _FULL2((1, DIM))]
    out_specs = [row((R, DIM))] * 3
    out_shape = [jax.ShapeDtypeStruct((L_SIZE, DIM), jnp.float32)] * 3
    return pl.pallas_call(body, grid=(nb,), in_specs=in_specs,
                          out_specs=out_specs, out_shape=out_shape)


def _init_msg(row64, W0, b0, W1, b1, W2, b2):
    def body(x, W0r, b0r, W1r, b1r, W2r, b2r, o):
        o[...] = _mlp3(x[...], W0r[...], b0r[...], W1r[...], b1r[...],
                       W2r[...], b2r[...])
    f = pl.pallas_call(
        body,
        out_shape=jax.ShapeDtypeStruct((8, DIM), jnp.float32))
    return f(jnp.broadcast_to(row64, (8, DIM)), W0, b0, W1, b1, W2, b2)


def _make_readout(R):
    nb = L_SIZE // R

    def body(h, W0, b0, W1, b1, W2, b2, o):
        s = _mlp3(h[...], W0[...], b0[...], W1[...], b1[...], W2[...], b2[...])
        sp = _swap_pairs(s)
        m = jnp.maximum(s, sp)
        e1 = jnp.exp(s - m)
        e2 = jnp.exp(sp - m)
        o[...] = e1 / (e1 + e2)

    row = lambda shape: pl.BlockSpec(shape, lambda i: (i, 0))
    in_specs = [row((R, DIM)),
                _FULL2((DIM, DIM)), _FULL2((1, DIM)),
                _FULL2((DIM, DIM)), _FULL2((1, DIM)),
                _FULL2((DIM, 1)), _FULL2((1, 1))]
    return pl.pallas_call(
        body, grid=(nb,), in_specs=in_specs, out_specs=row((R, 1)),
        out_shape=jax.ShapeDtypeStruct((L_SIZE, 1), jnp.float32))


def _prep_sorted(src, dst, n_q, qsize, e_pad):
    E = src.shape[0]
    pad = e_pad - E
    big = jnp.int32(n_q * qsize)
    dst_p = jnp.concatenate([dst, jnp.full((pad,), big, jnp.int32)])
    src_p = jnp.concatenate([src, jnp.zeros((pad,), jnp.int32)])
    order = jnp.argsort(dst_p)
    dst_s = dst_p[order]
    src_s = src_p[order]
    qid = dst_s // qsize
    dsts = tuple(
        jnp.where(qid == q, dst_s - q * qsize, qsize).reshape(-1, EB)
        for q in range(n_q))
    bounds = jnp.searchsorted(
        dst_s, (jnp.arange(n_q + 1) * qsize).astype(jnp.int32)).astype(jnp.int32)
    astart = (bounds[:-1] // SWEEP) * SWEEP
    aend = jnp.minimum(-(-bounds[1:] // SWEEP) * SWEEP, e_pad)
    aend = jnp.maximum(aend, astart)
    meta = jnp.stack([astart // EB, (aend - astart) // SWEEP], 1).reshape(-1)
    meta = jnp.concatenate(
        [meta, jnp.zeros((32 - 2 * n_q,), jnp.int32)]).astype(jnp.int32)
    return src_s.reshape(-1, EB), dsts, meta


def kernel(l_edge_index, c_edge_index, l_size, c_size, params):
    E = l_edge_index.shape[0]
    e_pad = -(-E // SWEEP) * SWEEP
    e_rows = e_pad // EB

    src_l2d, cdsts, meta_c = _prep_sorted(
        l_edge_index, c_edge_index, C_NQ, C_QSIZE, e_pad)
    src_c2d, ldsts, meta_l = _prep_sorted(
        c_edge_index, l_edge_index, L_NQ, L_QSIZE, e_pad)

    sc_l2c = _make_sc_aggregate(e_rows, L_SIZE, C_SIZE, C_NQ, C_QSIZE, C_QPAD)
    sc_c2l = _make_sc_aggregate(e_rows, C_SIZE, L_SIZE, L_NQ, L_QSIZE, L_QPAD)
    lstm_c = _make_lstm_c(2000)
    lstm_l = _make_lstm_l(2000)
    readout = _make_readout(2000)

    p = params
    cc, lc = p['c_cell'], p['l_cell']
    r2 = lambda v: v.reshape(1, -1)
    denom = math.sqrt(DIM)

    l_hidden = jnp.broadcast_to(p['l_init'] / denom, (L_SIZE, DIM))
    c_hidden = jnp.broadcast_to(p['c_init'] / denom, (C_SIZE, DIM))
    l_state = jnp.zeros((L_SIZE, DIM), jnp.float32)
    c_state = jnp.zeros((C_SIZE, DIM), jnp.float32)

    z64 = jnp.zeros((ZROWS, DIM), jnp.float32)

    mrow = _init_msg(p['l_init'] / denom, p['l2c_W'][0], r2(p['l2c_b'][0]),
                     p['l2c_W'][1], r2(p['l2c_b'][1]),
                     p['l2c_W'][2], r2(p['l2c_b'][2]))[0]
    msg_l = jnp.broadcast_to(mrow, (L_SIZE, DIM))

    for _ in range(N_ROUNDS):
        la = sc_l2c(src_l2d, *cdsts, meta_c, msg_l, z64)
        c_hidden, c_state, msg_c = lstm_c(
            la, c_hidden, c_state,
            cc['W_ih'], cc['W_hh'], r2(cc['g_ih']), r2(cc['b_ih']),
            r2(cc['g_hh']), r2(cc['b_hh']), r2(cc['g_c']), r2(cc['b_c']),
            p['c2l_W'][0], r2(p['c2l_b'][0]), p['c2l_W'][1], r2(p['c2l_b'][1]),
            p['c2l_W'][2], r2(p['c2l_b'][2]))
        ca = sc_c2l(src_c2d, *ldsts, meta_l, msg_c, z64)
        l_hidden, l_state, msg_l = lstm_l(
            ca, l_hidden, l_state,
            lc['W_ih'], lc['W_hh'], r2(lc['g_ih']), r2(lc['b_ih']),
            r2(lc['g_hh']), r2(lc['b_hh']), r2(lc['g_c']), r2(lc['b_c']),
            p['l2c_W'][0], r2(p['l2c_b'][0]), p['l2c_W'][1], r2(p['l2c_b'][1]),
            p['l2c_W'][2], r2(p['l2c_b'][2]))

    probs = readout(l_hidden, p['ro_W'][0], r2(p['ro_b'][0]),
                    p['ro_W'][1], r2(p['ro_b'][1]),
                    p['ro_W'][2], r2(p['ro_b'][2]))
    return probs.reshape(-1, 2)
